# baseline (speedup 1.0000x reference)
import functools

import jax
import jax.numpy as jnp
from jax import lax
from jax.experimental import pallas as pl
from jax.experimental.pallas import tpu as pltpu
from jax.experimental.pallas import tpu_sc as plsc

NUM_CLASSES_K = 100000
FEAT = 512
BATCH_K = 16384
LANES = 16
NC, NS = 2, 16
NW = NC * NS
ROWS_PER_W = BATCH_K // NW
CHUNK = 32
NCHUNK = ROWS_PER_W // CHUNK
VECS_PER_CHUNK = CHUNK * FEAT // LANES


def _make_sc_kernel():
    mesh = plsc.VectorSubcoreMesh(core_axis_name="c", subcore_axis_name="s")

    @functools.partial(
        pl.kernel,
        out_type=jax.ShapeDtypeStruct((NW, LANES), jnp.float32),
        mesh=mesh,
        scratch_types=[
            pltpu.VMEM((CHUNK,), jnp.int32),
            pltpu.VMEM((CHUNK, FEAT), jnp.float32),
            pltpu.VMEM((CHUNK, FEAT), jnp.float32),
            pltpu.VMEM((LANES,), jnp.float32),
            pltpu.SemaphoreType.DMA,
        ],
    )
    def k(feat_hbm, lab_hbm, cent_hbm, out_hbm, idx_v, rows_v, feat_v, acc_v, sem):
        wid = lax.axis_index("s") * NC + lax.axis_index("c")
        base = wid * ROWS_PER_W

        def chunk_body(kc, acc):
            off = base + kc * CHUNK
            pltpu.sync_copy(lab_hbm.at[pl.ds(off, CHUNK)], idx_v)
            gather = pltpu.make_async_copy(cent_hbm.at[idx_v], rows_v, sem)
            gather.start()
            pltpu.sync_copy(feat_hbm.at[pl.ds(off, CHUNK)], feat_v)
            gather.wait()

            def vec_body(t, a):
                r = t // (FEAT // LANES)
                j = t % (FEAT // LANES)
                f = feat_v[r, pl.ds(j * LANES, LANES)]
                c = rows_v[r, pl.ds(j * LANES, LANES)]
                d = f - c
                return a + d * d

            return lax.fori_loop(0, VECS_PER_CHUNK, vec_body, acc)

        acc = lax.fori_loop(0, NCHUNK, chunk_body, jnp.zeros((LANES,), jnp.float32))
        acc_v[...] = acc
        pltpu.sync_copy(acc_v, out_hbm.at[wid])

    return k


_sc_kernel = _make_sc_kernel()


def kernel(features, labels, centers):
    partials = _sc_kernel(features, labels.astype(jnp.int32), centers)
    batch = features.shape[0]
    return jnp.sum(partials) * (0.003 / batch)


# speedup vs baseline: 2.4193x; 2.4193x over previous
import functools

import jax
import jax.numpy as jnp
from jax import lax
from jax.experimental import pallas as pl
from jax.experimental.pallas import tpu as pltpu
from jax.experimental.pallas import tpu_sc as plsc

FEAT = 512
BATCH_K = 16384
LANES = 16
NC, NS = 2, 16
NW = NC * NS
ROWS_PER_W = BATCH_K // NW
CHUNK = 32
NCHUNK = ROWS_PER_W // CHUNK
VPR = FEAT // LANES


def _make_sc_kernel():
    mesh = plsc.VectorSubcoreMesh(core_axis_name="c", subcore_axis_name="s")

    @functools.partial(
        pl.kernel,
        out_type=jax.ShapeDtypeStruct((NW, LANES), jnp.float32),
        mesh=mesh,
        scratch_types=[
            pltpu.VMEM((NCHUNK, CHUNK), jnp.int32),
            pltpu.VMEM((CHUNK, FEAT), jnp.float32),
            pltpu.VMEM((CHUNK, FEAT), jnp.float32),
            pltpu.VMEM((CHUNK, FEAT), jnp.float32),
            pltpu.VMEM((CHUNK, FEAT), jnp.float32),
            pltpu.VMEM((LANES,), jnp.float32),
            pltpu.SemaphoreType.DMA,
            pltpu.SemaphoreType.DMA,
            pltpu.SemaphoreType.DMA,
            pltpu.SemaphoreType.DMA,
        ],
    )
    def k(feat_hbm, lab_hbm, cent_hbm, out_hbm,
          labs_v, rows0, rows1, feat0, feat1, acc_v,
          sg0, sg1, sf0, sf1):
        wid = lax.axis_index("s") * NC + lax.axis_index("c")
        base = wid * ROWS_PER_W
        rows_b = (rows0, rows1)
        feat_b = (feat0, feat1)
        sg = (sg0, sg1)
        sf = (sf0, sf1)

        pltpu.sync_copy(lab_hbm.at[pl.ds(wid * NCHUNK, NCHUNK)], labs_v)

        def start(kc, b):
            off = base + kc * CHUNK
            pltpu.make_async_copy(
                cent_hbm.at[labs_v.at[kc]], rows_b[b], sg[b]).start()
            pltpu.make_async_copy(
                feat_hbm.at[pl.ds(off, CHUNK)], feat_b[b], sf[b]).start()

        def wait(b):
            pltpu.make_async_copy(
                cent_hbm.at[labs_v.at[0]], rows_b[b], sg[b]).wait()
            pltpu.make_async_copy(
                feat_hbm.at[pl.ds(0, CHUNK)], feat_b[b], sf[b]).wait()

        def compute(b, acc):
            fb, rb = feat_b[b], rows_b[b]

            def row_body(r, accs):
                accs = list(accs)
                for j in range(VPR):
                    f = fb[r, pl.ds(j * LANES, LANES)]
                    c = rb[r, pl.ds(j * LANES, LANES)]
                    d = f - c
                    accs[j % 4] = accs[j % 4] + d * d
                return tuple(accs)

            return lax.fori_loop(0, CHUNK, row_body, acc)

        start(0, 0)
        zero = jnp.zeros((LANES,), jnp.float32)
        acc = (zero, zero, zero, zero)

        def step(k2, acc):
            for b in range(2):
                kc = k2 * 2 + b

                @pl.when(kc + 1 < NCHUNK)
                def _():
                    start(kc + 1, (b + 1) % 2)

                wait(b)
                acc = compute(b, acc)
            return acc

        acc = lax.fori_loop(0, NCHUNK // 2, step, acc)
        acc_v[...] = (acc[0] + acc[1]) + (acc[2] + acc[3])
        pltpu.sync_copy(acc_v, out_hbm.at[wid])

    return k


_sc_kernel = _make_sc_kernel()


def kernel(features, labels, centers):
    lab2d = labels.astype(jnp.int32).reshape(BATCH_K // CHUNK, CHUNK)
    partials = _sc_kernel(features, lab2d, centers)
    batch = features.shape[0]
    return jnp.sum(partials) * (0.003 / batch)
